# Initial kernel scaffold; baseline (speedup 1.0000x reference)
import jax
import jax.numpy as jnp
from jax.experimental import pallas as pl


def kernel(x, h, edge_index, W_ih, W_hh, b_ih, b_hh, W_gcn, b_gcn, W_lin, b_lin):
    raise NotImplementedError("write your pallas kernel here")


# speedup vs baseline: 119.8610x; 119.8610x over previous
import functools

import jax
import jax.numpy as jnp
from jax import lax
from jax.experimental import pallas as pl
from jax.experimental.pallas import tpu as pltpu
from jax.experimental.pallas import tpu_sc as plsc

_NC = 2
_NS = 16
_L = 16
_H = 24


def _pad_sizes(n):
    npt = -(-n // _NS)
    npt = ((npt + _L - 1) // _L) * _L
    return npt, npt * _NS


def _build_indeg(n, e):
    nw = _NC * _NS
    ew = e // nw
    c = 2000
    assert ew % c == 0 and e % nw == 0
    nch = ew // c
    npt, npad = _pad_sizes(n)
    mesh = plsc.VectorSubcoreMesh(core_axis_name="c", subcore_axis_name="s",
                                  num_cores=_NC, num_subcores=_NS)

    @functools.partial(
        pl.kernel,
        out_type=jax.ShapeDtypeStruct((_NC * npad,), jnp.float32),
        mesh=mesh,
        scratch_types=[
            pltpu.VMEM((c,), jnp.int32),
            pltpu.VMEM((c,), jnp.float32),
            pltpu.VMEM((npt,), jnp.float32),
            pltpu.VMEM_SHARED((npad,), jnp.float32),
        ],
    )
    def indeg(ei_ref, out_ref, idx_v, ones_v, zbuf_v, acc_sh):
        cid = lax.axis_index("c")
        sid = lax.axis_index("s")
        wid = sid * _NC + cid

        def fill_ones(i, carry):
            ones_v[pl.ds(i * _L, _L)] = jnp.ones((_L,), jnp.float32)
            return carry

        lax.fori_loop(0, c // _L, fill_ones, 0)

        def fill_zero(i, carry):
            zbuf_v[pl.ds(i * _L, _L)] = jnp.zeros((_L,), jnp.float32)
            return carry

        lax.fori_loop(0, npt // _L, fill_zero, 0)

        pltpu.sync_copy(zbuf_v, acc_sh.at[pl.ds(sid * npt, npt)])
        plsc.subcore_barrier()

        base = e + wid * ew

        def body(j, carry):
            pltpu.sync_copy(ei_ref.at[pl.ds(base + j * c, c)], idx_v)
            pltpu.sync_copy(ones_v, acc_sh.at[idx_v], add=True)
            return carry

        lax.fori_loop(0, nch, body, 0)
        plsc.subcore_barrier()
        pltpu.sync_copy(acc_sh.at[pl.ds(sid * npt, npt)], zbuf_v)
        pltpu.sync_copy(zbuf_v, out_ref.at[pl.ds(cid * npad + sid * npt, npt)])

    return indeg


def _build_segsum(n, e):
    nw = _NC * _NS
    ew = e // nw
    c = 2000
    assert ew % c == 0 and e % nw == 0
    nch = ew // c
    npt, npad = _pad_sizes(n)
    mesh = plsc.VectorSubcoreMesh(core_axis_name="c", subcore_axis_name="s",
                                  num_cores=_NC, num_subcores=_NS)

    @functools.partial(
        pl.kernel,
        out_type=jax.ShapeDtypeStruct((_NC * npad,), jnp.float32),
        mesh=mesh,
        scratch_types=[
            pltpu.VMEM((c,), jnp.int32),
            pltpu.VMEM((c,), jnp.int32),
            pltpu.VMEM((c,), jnp.float32),
            pltpu.VMEM((npt,), jnp.float32),
            pltpu.VMEM_SHARED((npad,), jnp.float32),
            pltpu.SemaphoreType.DMA,
        ],
    )
    def segsum(ei_ref, s_ref, out_ref, idx_r, idx_c, vals_v, zbuf_v, acc_sh, sem):
        cid = lax.axis_index("c")
        sid = lax.axis_index("s")
        wid = sid * _NC + cid

        def fill_zero(i, carry):
            zbuf_v[pl.ds(i * _L, _L)] = jnp.zeros((_L,), jnp.float32)
            return carry

        lax.fori_loop(0, npt // _L, fill_zero, 0)
        pltpu.sync_copy(zbuf_v, acc_sh.at[pl.ds(sid * npt, npt)])
        plsc.subcore_barrier()

        base = wid * ew

        def body(j, carry):
            pltpu.sync_copy(ei_ref.at[pl.ds(base + j * c, c)], idx_r)
            pltpu.sync_copy(ei_ref.at[pl.ds(e + base + j * c, c)], idx_c)
            pltpu.async_copy(s_ref.at[idx_r], vals_v, sem).wait()
            pltpu.sync_copy(vals_v, acc_sh.at[idx_c], add=True)
            return carry

        lax.fori_loop(0, nch, body, 0)
        plsc.subcore_barrier()
        pltpu.sync_copy(acc_sh.at[pl.ds(sid * npt, npt)], zbuf_v)
        pltpu.sync_copy(zbuf_v, out_ref.at[pl.ds(cid * npad + sid * npt, npt)])

    return segsum


def _lstm_body(x_ref, h0_ref, c0_ref, dp0_ref, dp1_ref, wih_ref, whh_ref,
               bih_ref, bhh_ref, wg_ref, h1_ref, c1_ref, s_ref, dis_ref):
    h0 = h0_ref[...]
    c0 = c0_ref[...]
    xv = x_ref[...]
    gates = lax.dot_general(h0, whh_ref[...], (((1,), (1,)), ((), ())),
                            preferred_element_type=jnp.float32)
    gates = gates + xv[:, None] * wih_ref[...] + bih_ref[...] + bhh_ref[...]
    ig = jax.nn.sigmoid(gates[:, 0:_H])
    fg = jax.nn.sigmoid(gates[:, _H:2 * _H])
    gg = jnp.tanh(gates[:, 2 * _H:3 * _H])
    og = jax.nn.sigmoid(gates[:, 3 * _H:4 * _H])
    c1 = fg * c0 + ig * gg
    h1 = og * jnp.tanh(c1)
    h1_ref[...] = h1
    c1_ref[...] = c1
    z = jnp.sum(h1 * wg_ref[...], axis=1)
    deg = dp0_ref[...] + dp1_ref[...] + 1.0
    dis = lax.rsqrt(deg)
    dis_ref[...] = dis
    s_ref[...] = dis * z


def _lstm_call(x, h0, c0, dp0, dp1, wih, whh, bih, bhh, wg, n):
    bn = 4096
    grid = -(-n // bn)
    f32 = jnp.float32
    return pl.pallas_call(
        _lstm_body,
        grid=(grid,),
        in_specs=[
            pl.BlockSpec((bn,), lambda i: (i,)),
            pl.BlockSpec((bn, _H), lambda i: (i, 0)),
            pl.BlockSpec((bn, _H), lambda i: (i, 0)),
            pl.BlockSpec((bn,), lambda i: (i,)),
            pl.BlockSpec((bn,), lambda i: (i,)),
            pl.BlockSpec((1, 4 * _H), lambda i: (0, 0)),
            pl.BlockSpec((4 * _H, _H), lambda i: (0, 0)),
            pl.BlockSpec((1, 4 * _H), lambda i: (0, 0)),
            pl.BlockSpec((1, 4 * _H), lambda i: (0, 0)),
            pl.BlockSpec((1, _H), lambda i: (0, 0)),
        ],
        out_specs=[
            pl.BlockSpec((bn, _H), lambda i: (i, 0)),
            pl.BlockSpec((bn, _H), lambda i: (i, 0)),
            pl.BlockSpec((bn,), lambda i: (i,)),
            pl.BlockSpec((bn,), lambda i: (i,)),
        ],
        out_shape=[
            jax.ShapeDtypeStruct((n, _H), f32),
            jax.ShapeDtypeStruct((n, _H), f32),
            jax.ShapeDtypeStruct((n,), f32),
            jax.ShapeDtypeStruct((n,), f32),
        ],
    )(x, h0, c0, dp0, dp1, wih, whh, bih, bhh, wg)


def _final_body(tp0_ref, tp1_ref, s_ref, dis_ref, x_ref, cc_ref, out_ref):
    t = tp0_ref[...] + tp1_ref[...] + s_ref[...]
    y = dis_ref[...] * t + cc_ref[0]
    out_ref[...] = x_ref[...] * y


def _final_call(tp0, tp1, s, dis, x, cc, n):
    bn = 4096
    grid = -(-n // bn)
    return pl.pallas_call(
        _final_body,
        grid=(grid,),
        in_specs=[
            pl.BlockSpec((bn,), lambda i: (i,)),
            pl.BlockSpec((bn,), lambda i: (i,)),
            pl.BlockSpec((bn,), lambda i: (i,)),
            pl.BlockSpec((bn,), lambda i: (i,)),
            pl.BlockSpec((bn,), lambda i: (i,)),
            pl.BlockSpec(memory_space=pltpu.SMEM),
        ],
        out_specs=pl.BlockSpec((bn,), lambda i: (i,)),
        out_shape=jax.ShapeDtypeStruct((n,), jnp.float32),
    )(tp0, tp1, s, dis, x, cc)


@jax.jit
def kernel(x, h, edge_index, W_ih, W_hh, b_ih, b_hh, W_gcn, b_gcn, W_lin, b_lin):
    n = x.shape[0]
    e = edge_index.shape[1]
    npt, npad = _pad_sizes(n)

    h0 = h[0, 0]
    c0 = h[1, 0]
    ei_flat = edge_index.reshape(-1)

    w = W_lin[0]
    wg = (W_gcn @ w).reshape(1, _H)
    cc = (b_gcn @ w + b_lin[0]).reshape(1)
    wih = W_ih.reshape(1, 4 * _H)
    bih = b_ih.reshape(1, 4 * _H)
    bhh = b_hh.reshape(1, 4 * _H)

    degp = _build_indeg(n, e)(ei_flat)
    dp0 = degp[:n]
    dp1 = degp[npad:npad + n]

    h1, c1, s, dis = _lstm_call(x, h0, c0, dp0, dp1, wih, W_hh, bih, bhh,
                                wg, n)

    tp = _build_segsum(n, e)(ei_flat, s)
    tp0 = tp[:n]
    tp1 = tp[npad:npad + n]

    out0 = _final_call(tp0, tp1, s, dis, x, cc, n)
    return (out0, h1[None, :, :], c1[None, :, :])


# speedup vs baseline: 120.6792x; 1.0068x over previous
import functools

import jax
import jax.numpy as jnp
from jax import lax
from jax.experimental import pallas as pl
from jax.experimental.pallas import tpu as pltpu
from jax.experimental.pallas import tpu_sc as plsc

_NC = 2
_NS = 16
_L = 16
_H = 24
_BN = 4096


def _pad_sizes(n):
    npt = -(-n // _NS)
    npt = ((npt + _L - 1) // _L) * _L
    npad = npt * _NS
    npad = ((npad + _BN - 1) // _BN) * _BN
    npt = npad // _NS
    return npt, npad


def _build_indeg(n, e):
    nw = _NC * _NS
    ew = e // nw
    c = 2000
    assert ew % c == 0 and e % nw == 0
    nch = ew // c
    npt, npad = _pad_sizes(n)
    mesh = plsc.VectorSubcoreMesh(core_axis_name="c", subcore_axis_name="s",
                                  num_cores=_NC, num_subcores=_NS)

    @functools.partial(
        pl.kernel,
        out_type=jax.ShapeDtypeStruct((_NC * npad,), jnp.float32),
        mesh=mesh,
        scratch_types=[
            pltpu.VMEM((c,), jnp.int32),
            pltpu.VMEM((c,), jnp.float32),
            pltpu.VMEM((npt,), jnp.float32),
            pltpu.VMEM_SHARED((npad,), jnp.float32),
        ],
    )
    def indeg(ei_ref, out_ref, idx_v, ones_v, zbuf_v, acc_sh):
        cid = lax.axis_index("c")
        sid = lax.axis_index("s")
        wid = sid * _NC + cid

        def fill_ones(i, carry):
            ones_v[pl.ds(i * _L, _L)] = jnp.ones((_L,), jnp.float32)
            return carry

        lax.fori_loop(0, c // _L, fill_ones, 0)

        def fill_zero(i, carry):
            zbuf_v[pl.ds(i * _L, _L)] = jnp.zeros((_L,), jnp.float32)
            return carry

        lax.fori_loop(0, npt // _L, fill_zero, 0)

        pltpu.sync_copy(zbuf_v, acc_sh.at[pl.ds(sid * npt, npt)])
        plsc.subcore_barrier()

        base = wid * ew

        def body(j, carry):
            pltpu.sync_copy(ei_ref.at[pl.ds(e + base + j * c, c)], idx_v)
            pltpu.sync_copy(ones_v, acc_sh.at[idx_v], add=True)
            return carry

        lax.fori_loop(0, nch, body, 0)
        plsc.subcore_barrier()
        pltpu.sync_copy(acc_sh.at[pl.ds(sid * npt, npt)], zbuf_v)
        pltpu.sync_copy(zbuf_v, out_ref.at[pl.ds(cid * npad + sid * npt, npt)])

    return indeg


def _build_segsum(n, e):
    nw = _NC * _NS
    ew = e // nw
    c = 2000
    assert ew % c == 0 and e % nw == 0
    nch = ew // c
    npt, npad = _pad_sizes(n)
    mesh = plsc.VectorSubcoreMesh(core_axis_name="c", subcore_axis_name="s",
                                  num_cores=_NC, num_subcores=_NS)

    @functools.partial(
        pl.kernel,
        out_type=jax.ShapeDtypeStruct((_NC * npad,), jnp.float32),
        mesh=mesh,
        scratch_types=[
            pltpu.VMEM((c,), jnp.int32),
            pltpu.VMEM((c,), jnp.int32),
            pltpu.VMEM((c,), jnp.float32),
            pltpu.VMEM((npt,), jnp.float32),
            pltpu.VMEM_SHARED((npad,), jnp.float32),
            pltpu.SemaphoreType.DMA,
        ],
    )
    def segsum(ei_ref, s_ref, out_ref, idx_r, idx_c, vals_v, zbuf_v, acc_sh,
               sem):
        cid = lax.axis_index("c")
        sid = lax.axis_index("s")
        wid = sid * _NC + cid

        def fill_zero(i, carry):
            zbuf_v[pl.ds(i * _L, _L)] = jnp.zeros((_L,), jnp.float32)
            return carry

        lax.fori_loop(0, npt // _L, fill_zero, 0)
        pltpu.sync_copy(zbuf_v, acc_sh.at[pl.ds(sid * npt, npt)])
        plsc.subcore_barrier()

        base = wid * ew

        def body(j, carry):
            pltpu.sync_copy(ei_ref.at[pl.ds(base + j * c, c)], idx_r)
            pltpu.sync_copy(ei_ref.at[pl.ds(e + base + j * c, c)], idx_c)
            pltpu.async_copy(s_ref.at[idx_r], vals_v, sem).wait()
            pltpu.sync_copy(vals_v, acc_sh.at[idx_c], add=True)
            return carry

        lax.fori_loop(0, nch, body, 0)
        plsc.subcore_barrier()
        pltpu.sync_copy(acc_sh.at[pl.ds(sid * npt, npt)], zbuf_v)
        pltpu.sync_copy(zbuf_v, out_ref.at[pl.ds(cid * npad + sid * npt, npt)])

    return segsum


def _lstm_body(x_ref, h0_ref, c0_ref, wih_ref, whh_ref, bih_ref, bhh_ref,
               wg_ref, h1_ref, c1_ref, z_ref):
    h0 = h0_ref[...]
    c0 = c0_ref[...]
    xv = x_ref[...]
    gates = lax.dot_general(h0, whh_ref[...], (((1,), (1,)), ((), ())),
                            preferred_element_type=jnp.float32)
    gates = gates + xv[:, None] * wih_ref[...] + bih_ref[...] + bhh_ref[...]
    ig = jax.nn.sigmoid(gates[:, 0:_H])
    fg = jax.nn.sigmoid(gates[:, _H:2 * _H])
    gg = jnp.tanh(gates[:, 2 * _H:3 * _H])
    og = jax.nn.sigmoid(gates[:, 3 * _H:4 * _H])
    c1 = fg * c0 + ig * gg
    h1 = og * jnp.tanh(c1)
    h1_ref[...] = h1
    c1_ref[...] = c1
    z_ref[...] = jnp.sum(h1 * wg_ref[...], axis=1)


def _lstm_call(x, h0, c0, wih, whh, bih, bhh, wg, n):
    grid = -(-n // _BN)
    f32 = jnp.float32
    return pl.pallas_call(
        _lstm_body,
        grid=(grid,),
        in_specs=[
            pl.BlockSpec((_BN,), lambda i: (i,)),
            pl.BlockSpec((_BN, _H), lambda i: (i, 0)),
            pl.BlockSpec((_BN, _H), lambda i: (i, 0)),
            pl.BlockSpec((1, 4 * _H), lambda i: (0, 0)),
            pl.BlockSpec((4 * _H, _H), lambda i: (0, 0)),
            pl.BlockSpec((1, 4 * _H), lambda i: (0, 0)),
            pl.BlockSpec((1, 4 * _H), lambda i: (0, 0)),
            pl.BlockSpec((1, _H), lambda i: (0, 0)),
        ],
        out_specs=[
            pl.BlockSpec((_BN, _H), lambda i: (i, 0)),
            pl.BlockSpec((_BN, _H), lambda i: (i, 0)),
            pl.BlockSpec((_BN,), lambda i: (i,)),
        ],
        out_shape=[
            jax.ShapeDtypeStruct((n, _H), f32),
            jax.ShapeDtypeStruct((n, _H), f32),
            jax.ShapeDtypeStruct((n,), f32),
        ],
    )(x, h0, c0, wih, whh, bih, bhh, wg)


def _scale_body(z_ref, dp0_ref, dp1_ref, dis_ref, s_ref):
    deg = dp0_ref[...] + dp1_ref[...] + 1.0
    dis = lax.rsqrt(deg)
    dis_ref[...] = dis
    s_ref[...] = dis * z_ref[...]


def _scale_call(z, degp, n, npad):
    grid = -(-n // _BN)
    off = npad // _BN
    return pl.pallas_call(
        _scale_body,
        grid=(grid,),
        in_specs=[
            pl.BlockSpec((_BN,), lambda i: (i,)),
            pl.BlockSpec((_BN,), lambda i: (i,)),
            pl.BlockSpec((_BN,), lambda i, off=off: (i + off,)),
        ],
        out_specs=[
            pl.BlockSpec((_BN,), lambda i: (i,)),
            pl.BlockSpec((_BN,), lambda i: (i,)),
        ],
        out_shape=[
            jax.ShapeDtypeStruct((n,), jnp.float32),
            jax.ShapeDtypeStruct((n,), jnp.float32),
        ],
    )(z, degp, degp)


def _final_body(tp0_ref, tp1_ref, s_ref, dis_ref, x_ref, cc_ref, out_ref):
    t = tp0_ref[...] + tp1_ref[...] + s_ref[...]
    y = dis_ref[...] * t + cc_ref[0]
    out_ref[...] = x_ref[...] * y


def _final_call(tp, s, dis, x, cc, n, npad):
    grid = -(-n // _BN)
    off = npad // _BN
    return pl.pallas_call(
        _final_body,
        grid=(grid,),
        in_specs=[
            pl.BlockSpec((_BN,), lambda i: (i,)),
            pl.BlockSpec((_BN,), lambda i, off=off: (i + off,)),
            pl.BlockSpec((_BN,), lambda i: (i,)),
            pl.BlockSpec((_BN,), lambda i: (i,)),
            pl.BlockSpec((_BN,), lambda i: (i,)),
            pl.BlockSpec(memory_space=pltpu.SMEM),
        ],
        out_specs=pl.BlockSpec((_BN,), lambda i: (i,)),
        out_shape=jax.ShapeDtypeStruct((n,), jnp.float32),
    )(tp, tp, s, dis, x, cc)


@jax.jit
def kernel(x, h, edge_index, W_ih, W_hh, b_ih, b_hh, W_gcn, b_gcn, W_lin, b_lin):
    n = x.shape[0]
    e = edge_index.shape[1]
    npt, npad = _pad_sizes(n)

    h0 = h[0, 0]
    c0 = h[1, 0]

    w = W_lin[0]
    wg = (W_gcn @ w).reshape(1, _H)
    cc = (b_gcn @ w + b_lin[0]).reshape(1)
    wih = W_ih.reshape(1, 4 * _H)
    bih = b_ih.reshape(1, 4 * _H)
    bhh = b_hh.reshape(1, 4 * _H)

    ei_flat = edge_index.reshape(-1)
    degp = _build_indeg(n, e)(ei_flat)
    h1, c1, z = _lstm_call(x, h0, c0, wih, W_hh, bih, bhh, wg, n)
    dis, s = _scale_call(z, degp, n, npad)
    tp = _build_segsum(n, e)(ei_flat, s)
    out0 = _final_call(tp, s, dis, x, cc, n, npad)
    return (out0, h1[None, :, :], c1[None, :, :])


# speedup vs baseline: 147.7443x; 1.2243x over previous
import functools

import jax
import jax.numpy as jnp
from jax import lax
from jax.experimental import pallas as pl
from jax.experimental.pallas import tpu as pltpu
from jax.experimental.pallas import tpu_sc as plsc

_NC = 2
_NS = 16
_L = 16
_H = 24
_BN = 4096


def _pad_sizes(n):
    npt = -(-n // _NS)
    npt = ((npt + _L - 1) // _L) * _L
    npad = npt * _NS
    npad = ((npad + _BN - 1) // _BN) * _BN
    npt = npad // _NS
    return npt, npad


def _build_indeg(n, e):
    nw = _NC * _NS
    ew = e // nw
    c = 2000
    assert ew % c == 0 and e % nw == 0
    nch = ew // c
    npt, npad = _pad_sizes(n)
    mesh = plsc.VectorSubcoreMesh(core_axis_name="c", subcore_axis_name="s",
                                  num_cores=_NC, num_subcores=_NS)

    @functools.partial(
        pl.kernel,
        out_type=jax.ShapeDtypeStruct((_NC * npad,), jnp.float32),
        mesh=mesh,
        scratch_types=[
            pltpu.VMEM((c,), jnp.int32),
            pltpu.VMEM((c,), jnp.float32),
            pltpu.VMEM((npt,), jnp.float32),
            pltpu.VMEM_SHARED((npad,), jnp.float32),
        ],
    )
    def indeg(ei_ref, out_ref, idx_v, ones_v, zbuf_v, acc_sh):
        cid = lax.axis_index("c")
        sid = lax.axis_index("s")
        wid = sid * _NC + cid

        def fill_ones(i, carry):
            ones_v[pl.ds(i * _L, _L)] = jnp.ones((_L,), jnp.float32)
            return carry

        lax.fori_loop(0, c // _L, fill_ones, 0)

        def fill_zero(i, carry):
            zbuf_v[pl.ds(i * _L, _L)] = jnp.zeros((_L,), jnp.float32)
            return carry

        lax.fori_loop(0, npt // _L, fill_zero, 0)

        pltpu.sync_copy(zbuf_v, acc_sh.at[pl.ds(sid * npt, npt)])
        plsc.subcore_barrier()

        base = wid * ew

        def body(j, carry):
            pltpu.sync_copy(ei_ref.at[pl.ds(e + base + j * c, c)], idx_v)
            pltpu.sync_copy(ones_v, acc_sh.at[idx_v], add=True)
            return carry

        lax.fori_loop(0, nch, body, 0)
        plsc.subcore_barrier()
        pltpu.sync_copy(acc_sh.at[pl.ds(sid * npt, npt)], zbuf_v)
        pltpu.sync_copy(zbuf_v, out_ref.at[pl.ds(cid * npad + sid * npt, npt)])

    return indeg


def _build_segsum(n, e):
    nw = _NC * _NS
    ew = e // nw
    c = 2000
    assert ew % c == 0 and e % nw == 0
    nch = ew // c
    npt, npad = _pad_sizes(n)
    mesh = plsc.VectorSubcoreMesh(core_axis_name="c", subcore_axis_name="s",
                                  num_cores=_NC, num_subcores=_NS)

    @functools.partial(
        pl.kernel,
        out_type=jax.ShapeDtypeStruct((_NC * npad,), jnp.float32),
        mesh=mesh,
        scratch_types=[
            pltpu.VMEM((c,), jnp.int32),
            pltpu.VMEM((c,), jnp.int32),
            pltpu.VMEM((c,), jnp.float32),
            pltpu.VMEM((npt,), jnp.float32),
            pltpu.VMEM_SHARED((npad,), jnp.float32),
            pltpu.VMEM_SHARED((npad,), jnp.float32),
        ],
    )
    def segsum(ei_ref, s_ref, out_ref, idx_r, idx_c, vals_v, zbuf_v, s_sh,
               acc_sh):
        cid = lax.axis_index("c")
        sid = lax.axis_index("s")
        wid = sid * _NC + cid

        def fill_zero(i, carry):
            zbuf_v[pl.ds(i * _L, _L)] = jnp.zeros((_L,), jnp.float32)
            return carry

        lax.fori_loop(0, npt // _L, fill_zero, 0)
        pltpu.sync_copy(zbuf_v, acc_sh.at[pl.ds(sid * npt, npt)])
        pltpu.sync_copy(s_ref.at[pl.ds(sid * npt, npt)], zbuf_v)
        pltpu.sync_copy(zbuf_v, s_sh.at[pl.ds(sid * npt, npt)])
        plsc.subcore_barrier()

        base = wid * ew

        def body(j, carry):
            pltpu.sync_copy(ei_ref.at[pl.ds(base + j * c, c)], idx_r)
            pltpu.sync_copy(ei_ref.at[pl.ds(e + base + j * c, c)], idx_c)
            pltpu.sync_copy(s_sh.at[idx_r], vals_v)
            pltpu.sync_copy(vals_v, acc_sh.at[idx_c], add=True)
            return carry

        lax.fori_loop(0, nch, body, 0)
        plsc.subcore_barrier()
        pltpu.sync_copy(acc_sh.at[pl.ds(sid * npt, npt)], zbuf_v)
        pltpu.sync_copy(zbuf_v, out_ref.at[pl.ds(cid * npad + sid * npt, npt)])

    return segsum


def _lstm_body(x_ref, h0_ref, c0_ref, wih_ref, whh_ref, bih_ref, bhh_ref,
               wg_ref, h1_ref, c1_ref, z_ref):
    h0 = h0_ref[...]
    c0 = c0_ref[...]
    xv = x_ref[...]
    gates = lax.dot_general(h0, whh_ref[...], (((1,), (1,)), ((), ())),
                            preferred_element_type=jnp.float32)
    gates = gates + xv[:, None] * wih_ref[...] + bih_ref[...] + bhh_ref[...]
    ig = jax.nn.sigmoid(gates[:, 0:_H])
    fg = jax.nn.sigmoid(gates[:, _H:2 * _H])
    gg = jnp.tanh(gates[:, 2 * _H:3 * _H])
    og = jax.nn.sigmoid(gates[:, 3 * _H:4 * _H])
    c1 = fg * c0 + ig * gg
    h1 = og * jnp.tanh(c1)
    h1_ref[...] = h1
    c1_ref[...] = c1
    z_ref[...] = jnp.sum(h1 * wg_ref[...], axis=1)


def _lstm_call(x, h0, c0, wih, whh, bih, bhh, wg, n):
    grid = -(-n // _BN)
    f32 = jnp.float32
    return pl.pallas_call(
        _lstm_body,
        grid=(grid,),
        in_specs=[
            pl.BlockSpec((_BN,), lambda i: (i,)),
            pl.BlockSpec((_BN, _H), lambda i: (i, 0)),
            pl.BlockSpec((_BN, _H), lambda i: (i, 0)),
            pl.BlockSpec((1, 4 * _H), lambda i: (0, 0)),
            pl.BlockSpec((4 * _H, _H), lambda i: (0, 0)),
            pl.BlockSpec((1, 4 * _H), lambda i: (0, 0)),
            pl.BlockSpec((1, 4 * _H), lambda i: (0, 0)),
            pl.BlockSpec((1, _H), lambda i: (0, 0)),
        ],
        out_specs=[
            pl.BlockSpec((_BN, _H), lambda i: (i, 0)),
            pl.BlockSpec((_BN, _H), lambda i: (i, 0)),
            pl.BlockSpec((_BN,), lambda i: (i,)),
        ],
        out_shape=[
            jax.ShapeDtypeStruct((n, _H), f32),
            jax.ShapeDtypeStruct((n, _H), f32),
            jax.ShapeDtypeStruct((n,), f32),
        ],
    )(x, h0, c0, wih, whh, bih, bhh, wg)


def _scale_body(z_ref, dp0_ref, dp1_ref, dis_ref, s_ref):
    deg = dp0_ref[...] + dp1_ref[...] + 1.0
    dis = lax.rsqrt(deg)
    dis_ref[...] = dis
    s_ref[...] = dis * z_ref[...]


def _scale_call(z, degp, n, npad):
    grid = npad // _BN
    off = npad // _BN
    return pl.pallas_call(
        _scale_body,
        grid=(grid,),
        in_specs=[
            pl.BlockSpec((_BN,), lambda i: (i,)),
            pl.BlockSpec((_BN,), lambda i: (i,)),
            pl.BlockSpec((_BN,), lambda i, off=off: (i + off,)),
        ],
        out_specs=[
            pl.BlockSpec((_BN,), lambda i: (i,)),
            pl.BlockSpec((_BN,), lambda i: (i,)),
        ],
        out_shape=[
            jax.ShapeDtypeStruct((n,), jnp.float32),
            jax.ShapeDtypeStruct((npad,), jnp.float32),
        ],
    )(z, degp, degp)


def _final_body(tp0_ref, tp1_ref, s_ref, dis_ref, x_ref, cc_ref, out_ref):
    t = tp0_ref[...] + tp1_ref[...] + s_ref[...]
    y = dis_ref[...] * t + cc_ref[0]
    out_ref[...] = x_ref[...] * y


def _final_call(tp, s, dis, x, cc, n, npad):
    grid = -(-n // _BN)
    off = npad // _BN
    return pl.pallas_call(
        _final_body,
        grid=(grid,),
        in_specs=[
            pl.BlockSpec((_BN,), lambda i: (i,)),
            pl.BlockSpec((_BN,), lambda i, off=off: (i + off,)),
            pl.BlockSpec((_BN,), lambda i: (i,)),
            pl.BlockSpec((_BN,), lambda i: (i,)),
            pl.BlockSpec((_BN,), lambda i: (i,)),
            pl.BlockSpec(memory_space=pltpu.SMEM),
        ],
        out_specs=pl.BlockSpec((_BN,), lambda i: (i,)),
        out_shape=jax.ShapeDtypeStruct((n,), jnp.float32),
    )(tp, tp, s, dis, x, cc)


@jax.jit
def kernel(x, h, edge_index, W_ih, W_hh, b_ih, b_hh, W_gcn, b_gcn, W_lin, b_lin):
    n = x.shape[0]
    e = edge_index.shape[1]
    npt, npad = _pad_sizes(n)

    h0 = h[0, 0]
    c0 = h[1, 0]

    w = W_lin[0]
    wg = (W_gcn @ w).reshape(1, _H)
    cc = (b_gcn @ w + b_lin[0]).reshape(1)
    wih = W_ih.reshape(1, 4 * _H)
    bih = b_ih.reshape(1, 4 * _H)
    bhh = b_hh.reshape(1, 4 * _H)

    ei_flat = edge_index.reshape(-1)
    degp = _build_indeg(n, e)(ei_flat)
    h1, c1, z = _lstm_call(x, h0, c0, wih, W_hh, bih, bhh, wg, n)
    dis, s = _scale_call(z, degp, n, npad)
    tp = _build_segsum(n, e)(ei_flat, s)
    out0 = _final_call(tp, s, dis, x, cc, n, npad)
    return (out0, h1[None, :, :], c1[None, :, :])


# speedup vs baseline: 197.4146x; 1.3362x over previous
import functools

import jax
import jax.numpy as jnp
from jax import lax
from jax.experimental import pallas as pl
from jax.experimental.pallas import tpu as pltpu
from jax.experimental.pallas import tpu_sc as plsc

_NC = 2
_NS = 16
_L = 16
_H = 24
_BN = 4096
_C = 10000


def _pad_sizes(n):
    npt = -(-n // _NS)
    npt = ((npt + _L - 1) // _L) * _L
    npad = npt * _NS
    npad = ((npad + _BN - 1) // _BN) * _BN
    npt = npad // _NS
    return npt, npad


def _fill_zero(ref, nwords):
    def body(i, carry):
        ref[pl.ds(i * _L, _L)] = jnp.zeros((_L,), jnp.float32)
        return carry

    lax.fori_loop(0, nwords // _L, body, 0)


def _build_indeg(n, e):
    nw = _NC * _NS
    ew = e // nw
    c = _C
    assert ew % c == 0 and e % nw == 0
    nch = ew // c
    assert nch % 2 == 0
    npt, npad = _pad_sizes(n)
    mesh = plsc.VectorSubcoreMesh(core_axis_name="c", subcore_axis_name="s",
                                  num_cores=_NC, num_subcores=_NS)

    @functools.partial(
        pl.kernel,
        out_type=jax.ShapeDtypeStruct((_NC * npad,), jnp.float32),
        mesh=mesh,
        scratch_types=[
            pltpu.VMEM((c,), jnp.int32),
            pltpu.VMEM((c,), jnp.int32),
            pltpu.VMEM((c,), jnp.float32),
            pltpu.VMEM((npt,), jnp.float32),
            pltpu.VMEM_SHARED((npad,), jnp.float32),
            pltpu.SemaphoreType.DMA,
            pltpu.SemaphoreType.DMA,
        ],
    )
    def indeg(ei_ref, out_ref, idx0, idx1, ones_v, zbuf_v, acc_sh, sem0, sem1):
        cid = lax.axis_index("c")
        sid = lax.axis_index("s")
        wid = sid * _NC + cid

        def fill_ones(i, carry):
            ones_v[pl.ds(i * _L, _L)] = jnp.ones((_L,), jnp.float32)
            return carry

        lax.fori_loop(0, c // _L, fill_ones, 0)
        _fill_zero(zbuf_v, npt)

        pltpu.sync_copy(zbuf_v, acc_sh.at[pl.ds(sid * npt, npt)])
        plsc.subcore_barrier()

        base = e + wid * ew
        bufs = (idx0, idx1)
        sems = (sem0, sem1)

        pltpu.async_copy(ei_ref.at[pl.ds(base, c)], idx0, sem0)

        def outer(jj, carry):
            for b in (0, 1):
                j = jj * 2 + b
                src = ei_ref.at[pl.ds(base + j * c, c)]
                pltpu.make_async_copy(src, bufs[b], sems[b]).wait()

                @pl.when(j + 1 < nch)
                def _():
                    nxt = ei_ref.at[pl.ds(base + (j + 1) * c, c)]
                    pltpu.async_copy(nxt, bufs[1 - b], sems[1 - b])

                pltpu.sync_copy(ones_v, acc_sh.at[bufs[b]], add=True)
            return carry

        lax.fori_loop(0, nch // 2, outer, 0)
        plsc.subcore_barrier()
        pltpu.sync_copy(acc_sh.at[pl.ds(sid * npt, npt)], zbuf_v)
        pltpu.sync_copy(zbuf_v, out_ref.at[pl.ds(cid * npad + sid * npt, npt)])

    return indeg


def _build_segsum(n, e):
    nw = _NC * _NS
    ew = e // nw
    c = _C
    assert ew % c == 0 and e % nw == 0
    nch = ew // c
    assert nch % 2 == 0
    npt, npad = _pad_sizes(n)
    mesh = plsc.VectorSubcoreMesh(core_axis_name="c", subcore_axis_name="s",
                                  num_cores=_NC, num_subcores=_NS)

    @functools.partial(
        pl.kernel,
        out_type=jax.ShapeDtypeStruct((_NC * npad,), jnp.float32),
        mesh=mesh,
        scratch_types=[
            pltpu.VMEM((c,), jnp.int32),
            pltpu.VMEM((c,), jnp.int32),
            pltpu.VMEM((c,), jnp.int32),
            pltpu.VMEM((c,), jnp.int32),
            pltpu.VMEM((c,), jnp.float32),
            pltpu.VMEM((npt,), jnp.float32),
            pltpu.VMEM_SHARED((npad,), jnp.float32),
            pltpu.VMEM_SHARED((npad,), jnp.float32),
            pltpu.SemaphoreType.DMA,
            pltpu.SemaphoreType.DMA,
            pltpu.SemaphoreType.DMA,
            pltpu.SemaphoreType.DMA,
        ],
    )
    def segsum(ei_ref, s_ref, out_ref, idxr0, idxr1, idxc0, idxc1, vals_v,
               zbuf_v, s_sh, acc_sh, semr0, semr1, semc0, semc1):
        cid = lax.axis_index("c")
        sid = lax.axis_index("s")
        wid = sid * _NC + cid

        _fill_zero(zbuf_v, npt)
        pltpu.sync_copy(zbuf_v, acc_sh.at[pl.ds(sid * npt, npt)])
        pltpu.sync_copy(s_ref.at[pl.ds(sid * npt, npt)], zbuf_v)
        pltpu.sync_copy(zbuf_v, s_sh.at[pl.ds(sid * npt, npt)])
        plsc.subcore_barrier()

        base = wid * ew
        rbufs = (idxr0, idxr1)
        cbufs = (idxc0, idxc1)
        rsems = (semr0, semr1)
        csems = (semc0, semc1)

        pltpu.async_copy(ei_ref.at[pl.ds(base, c)], idxr0, semr0)
        pltpu.async_copy(ei_ref.at[pl.ds(e + base, c)], idxc0, semc0)

        def outer(jj, carry):
            for b in (0, 1):
                j = jj * 2 + b
                rsrc = ei_ref.at[pl.ds(base + j * c, c)]
                csrc = ei_ref.at[pl.ds(e + base + j * c, c)]
                pltpu.make_async_copy(rsrc, rbufs[b], rsems[b]).wait()
                pltpu.make_async_copy(csrc, cbufs[b], csems[b]).wait()

                @pl.when(j + 1 < nch)
                def _():
                    rn = ei_ref.at[pl.ds(base + (j + 1) * c, c)]
                    cn = ei_ref.at[pl.ds(e + base + (j + 1) * c, c)]
                    pltpu.async_copy(rn, rbufs[1 - b], rsems[1 - b])
                    pltpu.async_copy(cn, cbufs[1 - b], csems[1 - b])

                pltpu.sync_copy(s_sh.at[rbufs[b]], vals_v)
                pltpu.sync_copy(vals_v, acc_sh.at[cbufs[b]], add=True)
            return carry

        lax.fori_loop(0, nch // 2, outer, 0)
        plsc.subcore_barrier()
        pltpu.sync_copy(acc_sh.at[pl.ds(sid * npt, npt)], zbuf_v)
        pltpu.sync_copy(zbuf_v, out_ref.at[pl.ds(cid * npad + sid * npt, npt)])

    return segsum


def _lstm_body(h_ref, x_ref, wih_ref, whh_ref, bih_ref, bhh_ref,
               wg_ref, h1_ref, c1_ref, z_ref):
    h0 = h_ref[0, 0]
    c0 = h_ref[1, 0]
    xv = x_ref[...]
    gates = lax.dot_general(h0, whh_ref[...], (((1,), (1,)), ((), ())),
                            preferred_element_type=jnp.float32)
    gates = gates + xv[:, None] * wih_ref[...] + bih_ref[...] + bhh_ref[...]
    ig = jax.nn.sigmoid(gates[:, 0:_H])
    fg = jax.nn.sigmoid(gates[:, _H:2 * _H])
    gg = jnp.tanh(gates[:, 2 * _H:3 * _H])
    og = jax.nn.sigmoid(gates[:, 3 * _H:4 * _H])
    c1 = fg * c0 + ig * gg
    h1 = og * jnp.tanh(c1)
    h1_ref[0] = h1
    c1_ref[0] = c1
    z_ref[...] = jnp.sum(h1 * wg_ref[...], axis=1)


def _lstm_call(h, x, wih, whh, bih, bhh, wg, n):
    grid = -(-n // _BN)
    f32 = jnp.float32
    return pl.pallas_call(
        _lstm_body,
        grid=(grid,),
        in_specs=[
            pl.BlockSpec((2, 1, _BN, _H), lambda i: (0, 0, i, 0)),
            pl.BlockSpec((_BN,), lambda i: (i,)),
            pl.BlockSpec((1, 4 * _H), lambda i: (0, 0)),
            pl.BlockSpec((4 * _H, _H), lambda i: (0, 0)),
            pl.BlockSpec((1, 4 * _H), lambda i: (0, 0)),
            pl.BlockSpec((1, 4 * _H), lambda i: (0, 0)),
            pl.BlockSpec((1, _H), lambda i: (0, 0)),
        ],
        out_specs=[
            pl.BlockSpec((1, _BN, _H), lambda i: (0, i, 0)),
            pl.BlockSpec((1, _BN, _H), lambda i: (0, i, 0)),
            pl.BlockSpec((_BN,), lambda i: (i,)),
        ],
        out_shape=[
            jax.ShapeDtypeStruct((1, n, _H), f32),
            jax.ShapeDtypeStruct((1, n, _H), f32),
            jax.ShapeDtypeStruct((n,), f32),
        ],
    )(h, x, wih, whh, bih, bhh, wg)


def _scale_body(z_ref, dp0_ref, dp1_ref, dis_ref, s_ref):
    deg = dp0_ref[...] + dp1_ref[...] + 1.0
    dis = lax.rsqrt(deg)
    dis_ref[...] = dis
    s_ref[...] = dis * z_ref[...]


def _scale_call(z, degp, n, npad):
    grid = npad // _BN
    off = npad // _BN
    return pl.pallas_call(
        _scale_body,
        grid=(grid,),
        in_specs=[
            pl.BlockSpec((_BN,), lambda i: (i,)),
            pl.BlockSpec((_BN,), lambda i: (i,)),
            pl.BlockSpec((_BN,), lambda i, off=off: (i + off,)),
        ],
        out_specs=[
            pl.BlockSpec((_BN,), lambda i: (i,)),
            pl.BlockSpec((_BN,), lambda i: (i,)),
        ],
        out_shape=[
            jax.ShapeDtypeStruct((n,), jnp.float32),
            jax.ShapeDtypeStruct((npad,), jnp.float32),
        ],
    )(z, degp, degp)


def _final_body(tp0_ref, tp1_ref, s_ref, dis_ref, x_ref, cc_ref, out_ref):
    t = tp0_ref[...] + tp1_ref[...] + s_ref[...]
    y = dis_ref[...] * t + cc_ref[0]
    out_ref[...] = x_ref[...] * y


def _final_call(tp, s, dis, x, cc, n, npad):
    grid = -(-n // _BN)
    off = npad // _BN
    return pl.pallas_call(
        _final_body,
        grid=(grid,),
        in_specs=[
            pl.BlockSpec((_BN,), lambda i: (i,)),
            pl.BlockSpec((_BN,), lambda i, off=off: (i + off,)),
            pl.BlockSpec((_BN,), lambda i: (i,)),
            pl.BlockSpec((_BN,), lambda i: (i,)),
            pl.BlockSpec((_BN,), lambda i: (i,)),
            pl.BlockSpec(memory_space=pltpu.SMEM),
        ],
        out_specs=pl.BlockSpec((_BN,), lambda i: (i,)),
        out_shape=jax.ShapeDtypeStruct((n,), jnp.float32),
    )(tp, tp, s, dis, x, cc)


@jax.jit
def kernel(x, h, edge_index, W_ih, W_hh, b_ih, b_hh, W_gcn, b_gcn, W_lin, b_lin):
    n = x.shape[0]
    e = edge_index.shape[1]
    npt, npad = _pad_sizes(n)

    w = W_lin[0]
    wg = (W_gcn @ w).reshape(1, _H)
    cc = (b_gcn @ w + b_lin[0]).reshape(1)
    wih = W_ih.reshape(1, 4 * _H)
    bih = b_ih.reshape(1, 4 * _H)
    bhh = b_hh.reshape(1, 4 * _H)

    ei_flat = edge_index.reshape(-1)
    degp = _build_indeg(n, e)(ei_flat)
    h1, c1, z = _lstm_call(h, x, wih, W_hh, bih, bhh, wg, n)
    dis, s = _scale_call(z, degp, n, npad)
    tp = _build_segsum(n, e)(ei_flat, s)
    out0 = _final_call(tp, s, dis, x, cc, n, npad)
    return (out0, h1, c1)


# speedup vs baseline: 394.5804x; 1.9987x over previous
import functools

import jax
import jax.numpy as jnp
from jax import lax
from jax.experimental import pallas as pl
from jax.experimental.pallas import tpu as pltpu
from jax.experimental.pallas import tpu_sc as plsc

_NC = 2
_NS = 16
_L = 16
_H = 24
_BN = 4096
_C = 10000


def _pad_sizes(n):
    npt = -(-n // _NS)
    npt = ((npt + _L - 1) // _L) * _L
    npad = npt * _NS
    npad = ((npad + _BN - 1) // _BN) * _BN
    npt = npad // _NS
    return npt, npad


def _fill_zero(ref, nwords):
    def body(i, carry):
        ref[pl.ds(i * _L, _L)] = jnp.zeros((_L,), jnp.float32)
        return carry

    lax.fori_loop(0, nwords // _L, body, 0)


def _build_indeg(n, e):
    nw = _NC * _NS
    ew = e // nw
    c = _C
    assert ew % c == 0 and e % nw == 0
    nch = ew // c
    assert nch % 2 == 0
    npt, npad = _pad_sizes(n)
    mesh = plsc.VectorSubcoreMesh(core_axis_name="c", subcore_axis_name="s",
                                  num_cores=_NC, num_subcores=_NS)

    @functools.partial(
        pl.kernel,
        out_type=jax.ShapeDtypeStruct((_NC * npad,), jnp.float32),
        mesh=mesh,
        scratch_types=[
            pltpu.VMEM((c,), jnp.int32),
            pltpu.VMEM((c,), jnp.int32),
            pltpu.VMEM((c,), jnp.float32),
            pltpu.VMEM((npt,), jnp.float32),
            pltpu.VMEM_SHARED((npad,), jnp.float32),
            pltpu.SemaphoreType.DMA,
            pltpu.SemaphoreType.DMA,
        ],
    )
    def indeg(ei_ref, out_ref, idx0, idx1, ones_v, zbuf_v, acc_sh, sem0, sem1):
        cid = lax.axis_index("c")
        sid = lax.axis_index("s")
        wid = sid * _NC + cid

        def fill_ones(i, carry):
            ones_v[pl.ds(i * _L, _L)] = jnp.ones((_L,), jnp.float32)
            return carry

        lax.fori_loop(0, c // _L, fill_ones, 0)
        _fill_zero(zbuf_v, npt)

        pltpu.sync_copy(zbuf_v, acc_sh.at[pl.ds(sid * npt, npt)])
        plsc.subcore_barrier()

        base = e + wid * ew
        bufs = (idx0, idx1)
        sems = (sem0, sem1)

        pltpu.async_copy(ei_ref.at[pl.ds(base, c)], idx0, sem0)

        def outer(jj, carry):
            for b in (0, 1):
                j = jj * 2 + b
                src = ei_ref.at[pl.ds(base + j * c, c)]
                pltpu.make_async_copy(src, bufs[b], sems[b]).wait()

                @pl.when(j + 1 < nch)
                def _():
                    nxt = ei_ref.at[pl.ds(base + (j + 1) * c, c)]
                    pltpu.async_copy(nxt, bufs[1 - b], sems[1 - b])

                pltpu.sync_copy(ones_v, acc_sh.at[bufs[b]], add=True)
            return carry

        lax.fori_loop(0, nch // 2, outer, 0)
        plsc.subcore_barrier()
        pltpu.sync_copy(acc_sh.at[pl.ds(sid * npt, npt)], zbuf_v)
        pltpu.sync_copy(zbuf_v, out_ref.at[pl.ds(cid * npad + sid * npt, npt)])

    return indeg


def _build_segsum(n, e):
    nw = _NC * _NS
    ew = e // nw
    c = _C
    assert ew % c == 0 and e % nw == 0
    nch = ew // c
    assert nch % 2 == 0
    npt, npad = _pad_sizes(n)
    mesh = plsc.VectorSubcoreMesh(core_axis_name="c", subcore_axis_name="s",
                                  num_cores=_NC, num_subcores=_NS)

    @functools.partial(
        pl.kernel,
        out_type=jax.ShapeDtypeStruct((_NC * npad,), jnp.float32),
        mesh=mesh,
        scratch_types=[
            pltpu.VMEM((c,), jnp.int32),
            pltpu.VMEM((c,), jnp.int32),
            pltpu.VMEM((c,), jnp.int32),
            pltpu.VMEM((c,), jnp.int32),
            pltpu.VMEM((c,), jnp.float32),
            pltpu.VMEM((npt,), jnp.float32),
            pltpu.VMEM_SHARED((npad,), jnp.float32),
            pltpu.VMEM_SHARED((npad,), jnp.float32),
            pltpu.SemaphoreType.DMA,
            pltpu.SemaphoreType.DMA,
            pltpu.SemaphoreType.DMA,
            pltpu.SemaphoreType.DMA,
        ],
    )
    def segsum(ei_ref, s_ref, out_ref, idxr0, idxr1, idxc0, idxc1, vals_v,
               zbuf_v, s_sh, acc_sh, semr0, semr1, semc0, semc1):
        cid = lax.axis_index("c")
        sid = lax.axis_index("s")
        wid = sid * _NC + cid

        _fill_zero(zbuf_v, npt)
        pltpu.sync_copy(zbuf_v, acc_sh.at[pl.ds(sid * npt, npt)])
        pltpu.sync_copy(s_ref.at[pl.ds(sid * npt, npt)], zbuf_v)
        pltpu.sync_copy(zbuf_v, s_sh.at[pl.ds(sid * npt, npt)])
        plsc.subcore_barrier()

        base = wid * ew
        rbufs = (idxr0, idxr1)
        cbufs = (idxc0, idxc1)
        rsems = (semr0, semr1)
        csems = (semc0, semc1)

        pltpu.async_copy(ei_ref.at[pl.ds(base, c)], idxr0, semr0)
        pltpu.async_copy(ei_ref.at[pl.ds(e + base, c)], idxc0, semc0)

        def outer(jj, carry):
            for b in (0, 1):
                j = jj * 2 + b
                rsrc = ei_ref.at[pl.ds(base + j * c, c)]
                csrc = ei_ref.at[pl.ds(e + base + j * c, c)]
                pltpu.make_async_copy(rsrc, rbufs[b], rsems[b]).wait()
                pltpu.make_async_copy(csrc, cbufs[b], csems[b]).wait()

                @pl.when(j + 1 < nch)
                def _():
                    rn = ei_ref.at[pl.ds(base + (j + 1) * c, c)]
                    cn = ei_ref.at[pl.ds(e + base + (j + 1) * c, c)]
                    pltpu.async_copy(rn, rbufs[1 - b], rsems[1 - b])
                    pltpu.async_copy(cn, cbufs[1 - b], csems[1 - b])

                pltpu.sync_copy(s_sh.at[rbufs[b]], vals_v)
                pltpu.sync_copy(vals_v, acc_sh.at[cbufs[b]], add=True)
            return carry

        lax.fori_loop(0, nch // 2, outer, 0)
        plsc.subcore_barrier()
        pltpu.sync_copy(acc_sh.at[pl.ds(sid * npt, npt)], zbuf_v)
        pltpu.sync_copy(zbuf_v, out_ref.at[pl.ds(cid * npad + sid * npt, npt)])

    return segsum


def _lstm_body(ht_ref, x_ref, whx_ref, wg_ref, h1_ref, c1_ref, z_ref):
    h0 = ht_ref[0, 0]
    c0 = ht_ref[1, 0]
    xv = x_ref[...]
    ones = jnp.ones((1, xv.shape[0]), jnp.float32)
    hx = jnp.concatenate([h0, xv[None, :], ones], axis=0)
    gates = lax.dot_general(whx_ref[...], hx, (((1,), (0,)), ((), ())),
                            preferred_element_type=jnp.float32)
    ig = jax.nn.sigmoid(gates[0:_H])
    fg = jax.nn.sigmoid(gates[_H:2 * _H])
    gg = jnp.tanh(gates[2 * _H:3 * _H])
    og = jax.nn.sigmoid(gates[3 * _H:4 * _H])
    c1 = fg * c0 + ig * gg
    h1 = og * jnp.tanh(c1)
    h1_ref[0] = h1
    c1_ref[0] = c1
    zm = lax.dot_general(wg_ref[...], h1, (((1,), (0,)), ((), ())),
                         preferred_element_type=jnp.float32)
    z_ref[...] = zm[0]


def _lstm_call(ht, x, whx, wg, n):
    grid = -(-n // _BN)
    f32 = jnp.float32
    return pl.pallas_call(
        _lstm_body,
        grid=(grid,),
        in_specs=[
            pl.BlockSpec((2, 1, _H, _BN), lambda i: (0, 0, 0, i)),
            pl.BlockSpec((_BN,), lambda i: (i,)),
            pl.BlockSpec((4 * _H, _H + 2), lambda i: (0, 0)),
            pl.BlockSpec((1, _H), lambda i: (0, 0)),
        ],
        out_specs=[
            pl.BlockSpec((1, _H, _BN), lambda i: (0, 0, i)),
            pl.BlockSpec((1, _H, _BN), lambda i: (0, 0, i)),
            pl.BlockSpec((_BN,), lambda i: (i,)),
        ],
        out_shape=[
            jax.ShapeDtypeStruct((1, _H, n), f32),
            jax.ShapeDtypeStruct((1, _H, n), f32),
            jax.ShapeDtypeStruct((n,), f32),
        ],
    )(ht, x, whx, wg)


def _scale_body(z_ref, dp0_ref, dp1_ref, dis_ref, s_ref):
    deg = dp0_ref[...] + dp1_ref[...] + 1.0
    dis = lax.rsqrt(deg)
    dis_ref[...] = dis
    s_ref[...] = dis * z_ref[...]


def _scale_call(z, degp, n, npad):
    grid = npad // _BN
    off = npad // _BN
    return pl.pallas_call(
        _scale_body,
        grid=(grid,),
        in_specs=[
            pl.BlockSpec((_BN,), lambda i: (i,)),
            pl.BlockSpec((_BN,), lambda i: (i,)),
            pl.BlockSpec((_BN,), lambda i, off=off: (i + off,)),
        ],
        out_specs=[
            pl.BlockSpec((_BN,), lambda i: (i,)),
            pl.BlockSpec((_BN,), lambda i: (i,)),
        ],
        out_shape=[
            jax.ShapeDtypeStruct((n,), jnp.float32),
            jax.ShapeDtypeStruct((npad,), jnp.float32),
        ],
    )(z, degp, degp)


def _final_body(tp0_ref, tp1_ref, s_ref, dis_ref, x_ref, cc_ref, out_ref):
    t = tp0_ref[...] + tp1_ref[...] + s_ref[...]
    y = dis_ref[...] * t + cc_ref[0]
    out_ref[...] = x_ref[...] * y


def _final_call(tp, s, dis, x, cc, n, npad):
    grid = -(-n // _BN)
    off = npad // _BN
    return pl.pallas_call(
        _final_body,
        grid=(grid,),
        in_specs=[
            pl.BlockSpec((_BN,), lambda i: (i,)),
            pl.BlockSpec((_BN,), lambda i, off=off: (i + off,)),
            pl.BlockSpec((_BN,), lambda i: (i,)),
            pl.BlockSpec((_BN,), lambda i: (i,)),
            pl.BlockSpec((_BN,), lambda i: (i,)),
            pl.BlockSpec(memory_space=pltpu.SMEM),
        ],
        out_specs=pl.BlockSpec((_BN,), lambda i: (i,)),
        out_shape=jax.ShapeDtypeStruct((n,), jnp.float32),
    )(tp, tp, s, dis, x, cc)


@jax.jit
def kernel(x, h, edge_index, W_ih, W_hh, b_ih, b_hh, W_gcn, b_gcn, W_lin, b_lin):
    n = x.shape[0]
    e = edge_index.shape[1]
    npt, npad = _pad_sizes(n)

    w = W_lin[0]
    wg = (W_gcn @ w).reshape(1, _H)
    cc = (b_gcn @ w + b_lin[0]).reshape(1)
    whx = jnp.concatenate(
        [W_hh, W_ih, (b_ih + b_hh).reshape(4 * _H, 1)], axis=1)

    ht = jnp.transpose(h, (0, 1, 3, 2))
    ei_flat = edge_index.reshape(-1)
    degp = _build_indeg(n, e)(ei_flat)
    h1t, c1t, z = _lstm_call(ht, x, whx, wg, n)
    dis, s = _scale_call(z, degp, n, npad)
    tp = _build_segsum(n, e)(ei_flat, s)
    out0 = _final_call(tp, s, dis, x, cc, n, npad)
    return (out0, jnp.transpose(h1t, (0, 2, 1)), jnp.transpose(c1t, (0, 2, 1)))


# speedup vs baseline: 444.1288x; 1.1256x over previous
import functools

import jax
import jax.numpy as jnp
from jax import lax
from jax.experimental import pallas as pl
from jax.experimental.pallas import tpu as pltpu
from jax.experimental.pallas import tpu_sc as plsc

_NC = 2
_NS = 16
_L = 16
_H = 24
_BN = 4096
_BNL = 8192
_BN2 = 25600
_C = 10000


def _pad_sizes(n):
    npt = -(-n // _NS)
    npt = ((npt + _L - 1) // _L) * _L
    npad = npt * _NS
    npad = ((npad + _BN - 1) // _BN) * _BN
    npt = npad // _NS
    return npt, npad


def _fill_zero(ref, nwords):
    def body(i, carry):
        ref[pl.ds(i * _L, _L)] = jnp.zeros((_L,), jnp.float32)
        return carry

    lax.fori_loop(0, nwords // _L, body, 0)


def _build_indeg(n, e):
    nw = _NC * _NS
    ew = e // nw
    c = _C
    assert ew % c == 0 and e % nw == 0
    nch = ew // c
    assert nch % 2 == 0
    npt, npad = _pad_sizes(n)
    mesh = plsc.VectorSubcoreMesh(core_axis_name="c", subcore_axis_name="s",
                                  num_cores=_NC, num_subcores=_NS)

    @functools.partial(
        pl.kernel,
        out_type=jax.ShapeDtypeStruct((_NC * npad,), jnp.float32),
        mesh=mesh,
        scratch_types=[
            pltpu.VMEM((c,), jnp.int32),
            pltpu.VMEM((c,), jnp.int32),
            pltpu.VMEM((c,), jnp.float32),
            pltpu.VMEM((npt,), jnp.float32),
            pltpu.VMEM_SHARED((npad,), jnp.float32),
            pltpu.SemaphoreType.DMA,
            pltpu.SemaphoreType.DMA,
        ],
    )
    def indeg(ei_ref, out_ref, idx0, idx1, ones_v, zbuf_v, acc_sh, sem0, sem1):
        cid = lax.axis_index("c")
        sid = lax.axis_index("s")
        wid = sid * _NC + cid

        def fill_ones(i, carry):
            ones_v[pl.ds(i * _L, _L)] = jnp.ones((_L,), jnp.float32)
            return carry

        lax.fori_loop(0, c // _L, fill_ones, 0)
        _fill_zero(zbuf_v, npt)

        pltpu.sync_copy(zbuf_v, acc_sh.at[pl.ds(sid * npt, npt)])
        plsc.subcore_barrier()

        base = e + wid * ew
        bufs = (idx0, idx1)
        sems = (sem0, sem1)

        pltpu.async_copy(ei_ref.at[pl.ds(base, c)], idx0, sem0)

        def outer(jj, carry):
            for b in (0, 1):
                j = jj * 2 + b
                src = ei_ref.at[pl.ds(base + j * c, c)]
                pltpu.make_async_copy(src, bufs[b], sems[b]).wait()

                @pl.when(j + 1 < nch)
                def _():
                    nxt = ei_ref.at[pl.ds(base + (j + 1) * c, c)]
                    pltpu.async_copy(nxt, bufs[1 - b], sems[1 - b])

                pltpu.sync_copy(ones_v, acc_sh.at[bufs[b]], add=True)
            return carry

        lax.fori_loop(0, nch // 2, outer, 0)
        plsc.subcore_barrier()
        pltpu.sync_copy(acc_sh.at[pl.ds(sid * npt, npt)], zbuf_v)
        pltpu.sync_copy(zbuf_v, out_ref.at[pl.ds(cid * npad + sid * npt, npt)])

    return indeg


def _build_segsum(n, e):
    nw = _NC * _NS
    ew = e // nw
    c = _C
    assert ew % c == 0 and e % nw == 0
    nch = ew // c
    assert nch % 2 == 0
    npt, npad = _pad_sizes(n)
    mesh = plsc.VectorSubcoreMesh(core_axis_name="c", subcore_axis_name="s",
                                  num_cores=_NC, num_subcores=_NS)

    @functools.partial(
        pl.kernel,
        out_type=jax.ShapeDtypeStruct((_NC * npad,), jnp.float32),
        mesh=mesh,
        scratch_types=[
            pltpu.VMEM((c,), jnp.int32),
            pltpu.VMEM((c,), jnp.int32),
            pltpu.VMEM((c,), jnp.int32),
            pltpu.VMEM((c,), jnp.int32),
            pltpu.VMEM((c,), jnp.float32),
            pltpu.VMEM((npt,), jnp.float32),
            pltpu.VMEM_SHARED((npad,), jnp.float32),
            pltpu.VMEM_SHARED((npad,), jnp.float32),
            pltpu.SemaphoreType.DMA,
            pltpu.SemaphoreType.DMA,
            pltpu.SemaphoreType.DMA,
            pltpu.SemaphoreType.DMA,
        ],
    )
    def segsum(ei_ref, s_ref, out_ref, idxr0, idxr1, idxc0, idxc1, vals_v,
               zbuf_v, s_sh, acc_sh, semr0, semr1, semc0, semc1):
        cid = lax.axis_index("c")
        sid = lax.axis_index("s")
        wid = sid * _NC + cid

        _fill_zero(zbuf_v, npt)
        pltpu.sync_copy(zbuf_v, acc_sh.at[pl.ds(sid * npt, npt)])
        pltpu.sync_copy(s_ref.at[pl.ds(sid * npt, npt)], zbuf_v)
        pltpu.sync_copy(zbuf_v, s_sh.at[pl.ds(sid * npt, npt)])
        plsc.subcore_barrier()

        base = wid * ew
        rbufs = (idxr0, idxr1)
        cbufs = (idxc0, idxc1)
        rsems = (semr0, semr1)
        csems = (semc0, semc1)

        pltpu.async_copy(ei_ref.at[pl.ds(base, c)], idxr0, semr0)
        pltpu.async_copy(ei_ref.at[pl.ds(e + base, c)], idxc0, semc0)

        def outer(jj, carry):
            for b in (0, 1):
                j = jj * 2 + b
                rsrc = ei_ref.at[pl.ds(base + j * c, c)]
                csrc = ei_ref.at[pl.ds(e + base + j * c, c)]
                pltpu.make_async_copy(rsrc, rbufs[b], rsems[b]).wait()
                pltpu.make_async_copy(csrc, cbufs[b], csems[b]).wait()

                @pl.when(j + 1 < nch)
                def _():
                    rn = ei_ref.at[pl.ds(base + (j + 1) * c, c)]
                    cn = ei_ref.at[pl.ds(e + base + (j + 1) * c, c)]
                    pltpu.async_copy(rn, rbufs[1 - b], rsems[1 - b])
                    pltpu.async_copy(cn, cbufs[1 - b], csems[1 - b])

                pltpu.sync_copy(s_sh.at[rbufs[b]], vals_v)
                pltpu.sync_copy(vals_v, acc_sh.at[cbufs[b]], add=True)
            return carry

        lax.fori_loop(0, nch // 2, outer, 0)
        plsc.subcore_barrier()
        pltpu.sync_copy(acc_sh.at[pl.ds(sid * npt, npt)], zbuf_v)
        pltpu.sync_copy(zbuf_v, out_ref.at[pl.ds(cid * npad + sid * npt, npt)])

    return segsum


def _lstm_body(ht_ref, x_ref, whx_ref, wg_ref, h1_ref, c1_ref, z_ref):
    h0 = ht_ref[0, 0]
    c0 = ht_ref[1, 0]
    xv = x_ref[...]
    ones = jnp.ones((1, xv.shape[0]), jnp.float32)
    hx = jnp.concatenate([h0, xv[None, :], ones], axis=0)
    gates = lax.dot_general(whx_ref[...], hx, (((1,), (0,)), ((), ())),
                            preferred_element_type=jnp.float32)
    ig = jax.nn.sigmoid(gates[0:_H])
    fg = jax.nn.sigmoid(gates[_H:2 * _H])
    gg = jnp.tanh(gates[2 * _H:3 * _H])
    og = jax.nn.sigmoid(gates[3 * _H:4 * _H])
    c1 = fg * c0 + ig * gg
    h1 = og * jnp.tanh(c1)
    h1_ref[0] = h1
    c1_ref[0] = c1
    zm = lax.dot_general(wg_ref[...], h1, (((1,), (0,)), ((), ())),
                         preferred_element_type=jnp.float32)
    z_ref[...] = zm[0]


def _lstm_call(ht, x, whx, wg, n):
    grid = -(-n // _BNL)
    f32 = jnp.float32
    return pl.pallas_call(
        _lstm_body,
        grid=(grid,),
        in_specs=[
            pl.BlockSpec((2, 1, _H, _BNL), lambda i: (0, 0, 0, i)),
            pl.BlockSpec((_BNL,), lambda i: (i,)),
            pl.BlockSpec((4 * _H, _H + 2), lambda i: (0, 0)),
            pl.BlockSpec((1, _H), lambda i: (0, 0)),
        ],
        out_specs=[
            pl.BlockSpec((1, _H, _BNL), lambda i: (0, 0, i)),
            pl.BlockSpec((1, _H, _BNL), lambda i: (0, 0, i)),
            pl.BlockSpec((_BNL,), lambda i: (i,)),
        ],
        out_shape=[
            jax.ShapeDtypeStruct((1, _H, n), f32),
            jax.ShapeDtypeStruct((1, _H, n), f32),
            jax.ShapeDtypeStruct((n,), f32),
        ],
    )(ht, x, whx, wg)


def _scale_body(z_ref, dp0_ref, dp1_ref, dis_ref, s_ref):
    deg = dp0_ref[...] + dp1_ref[...] + 1.0
    dis = lax.rsqrt(deg)
    dis_ref[...] = dis
    s_ref[...] = dis * z_ref[...]


def _scale_call(z, degp, n, npad):
    grid = npad // _BN2
    off = npad // _BN2
    return pl.pallas_call(
        _scale_body,
        grid=(grid,),
        in_specs=[
            pl.BlockSpec((_BN2,), lambda i: (i,)),
            pl.BlockSpec((_BN2,), lambda i: (i,)),
            pl.BlockSpec((_BN2,), lambda i, off=off: (i + off,)),
        ],
        out_specs=[
            pl.BlockSpec((_BN2,), lambda i: (i,)),
            pl.BlockSpec((_BN2,), lambda i: (i,)),
        ],
        out_shape=[
            jax.ShapeDtypeStruct((n,), jnp.float32),
            jax.ShapeDtypeStruct((npad,), jnp.float32),
        ],
    )(z, degp, degp)


def _final_body(tp0_ref, tp1_ref, s_ref, dis_ref, x_ref, cc_ref, out_ref):
    t = tp0_ref[...] + tp1_ref[...] + s_ref[...]
    y = dis_ref[...] * t + cc_ref[0]
    out_ref[...] = x_ref[...] * y


def _final_call(tp, s, dis, x, cc, n, npad):
    grid = -(-n // _BN2)
    off = npad // _BN2
    return pl.pallas_call(
        _final_body,
        grid=(grid,),
        in_specs=[
            pl.BlockSpec((_BN2,), lambda i: (i,)),
            pl.BlockSpec((_BN2,), lambda i, off=off: (i + off,)),
            pl.BlockSpec((_BN2,), lambda i: (i,)),
            pl.BlockSpec((_BN2,), lambda i: (i,)),
            pl.BlockSpec((_BN2,), lambda i: (i,)),
            pl.BlockSpec(memory_space=pltpu.SMEM),
        ],
        out_specs=pl.BlockSpec((_BN2,), lambda i: (i,)),
        out_shape=jax.ShapeDtypeStruct((n,), jnp.float32),
    )(tp, tp, s, dis, x, cc)


@jax.jit
def kernel(x, h, edge_index, W_ih, W_hh, b_ih, b_hh, W_gcn, b_gcn, W_lin, b_lin):
    n = x.shape[0]
    e = edge_index.shape[1]
    npt, npad = _pad_sizes(n)

    w = W_lin[0]
    wg = (W_gcn @ w).reshape(1, _H)
    cc = (b_gcn @ w + b_lin[0]).reshape(1)
    whx = jnp.concatenate(
        [W_hh, W_ih, (b_ih + b_hh).reshape(4 * _H, 1)], axis=1)

    ht = jnp.transpose(h, (0, 1, 3, 2))
    ei_flat = edge_index.reshape(-1)
    degp = _build_indeg(n, e)(ei_flat)
    h1t, c1t, z = _lstm_call(ht, x, whx, wg, n)
    dis, s = _scale_call(z, degp, n, npad)
    tp = _build_segsum(n, e)(ei_flat, s)
    out0 = _final_call(tp, s, dis, x, cc, n, npad)
    return (out0, jnp.transpose(h1t, (0, 2, 1)), jnp.transpose(c1t, (0, 2, 1)))
